# Initial kernel scaffold; baseline (speedup 1.0000x reference)
import jax
import jax.numpy as jnp
from jax.experimental import pallas as pl


def kernel(token_ids, embedding_matrix):
    raise NotImplementedError("write your pallas kernel here")


# speedup vs baseline: 1.6833x; 1.6833x over previous
import functools

import jax
import jax.numpy as jnp
from jax import lax
from jax.experimental import pallas as pl
from jax.experimental.pallas import tpu as pltpu
from jax.experimental.pallas import tpu_sc as plsc

NC = 2
NS = 16
NW = NC * NS

EMB_DIM = 64
CHUNK = 128


def _make_gather(num_rows: int, n_chunks: int):
  mesh = plsc.VectorSubcoreMesh(core_axis_name="c", subcore_axis_name="s")

  @functools.partial(
      pl.kernel,
      out_type=jax.ShapeDtypeStruct((num_rows, EMB_DIM), jnp.float32),
      mesh=mesh,
      scratch_types=[
          pltpu.VMEM((n_chunks, CHUNK), jnp.int32),
          pltpu.VMEM((CHUNK, EMB_DIM), jnp.float32),
          pltpu.SemaphoreType.DMA,
      ],
      compiler_params=pltpu.CompilerParams(use_tc_tiling_on_sc=False),
  )
  def gather_kernel(table_hbm, idx_hbm, out_hbm, idx_v, rows_v, sem):
    wid = lax.axis_index("s") * NC + lax.axis_index("c")
    base = wid * (n_chunks * CHUNK)
    pltpu.sync_copy(idx_hbm.at[wid], idx_v)

    def chunk_body(g, _):
      pltpu.async_copy(table_hbm.at[idx_v.at[g]], rows_v, sem).wait()
      pltpu.sync_copy(rows_v, out_hbm.at[pl.ds(base + g * CHUNK, CHUNK)])
      return 0

    lax.fori_loop(0, n_chunks, chunk_body, 0)

  return gather_kernel


def kernel(token_ids, embedding_matrix):
  batch, hist = token_ids.shape
  total = batch * hist
  per_w = total // NW
  n_chunks = per_w // CHUNK
  idx = token_ids.astype(jnp.int32).reshape(NW, n_chunks, CHUNK)
  out = _make_gather(total, n_chunks)(embedding_matrix, idx)
  return out.reshape(batch, hist, EMB_DIM)


# speedup vs baseline: 1.8731x; 1.1127x over previous
import functools

import jax
import jax.numpy as jnp
from jax import lax
from jax.experimental import pallas as pl
from jax.experimental.pallas import tpu as pltpu
from jax.experimental.pallas import tpu_sc as plsc

NC = 2
NS = 16
NW = NC * NS

EMB_DIM = 64
CHUNK = 128
K = 2
SUPER = K * CHUNK
NBUF = 4


def _make_gather(num_rows: int, n_chunks: int):
  n_super = n_chunks // K
  n_groups = n_super // NBUF
  per_w = n_chunks * CHUNK
  mesh = plsc.VectorSubcoreMesh(core_axis_name="c", subcore_axis_name="s")

  scratch = [pltpu.VMEM((n_chunks, CHUNK), jnp.int32)]
  scratch += [pltpu.VMEM((SUPER, EMB_DIM), jnp.float32) for _ in range(NBUF)]
  scratch += [pltpu.SemaphoreType.DMA for _ in range(2 * NBUF)]

  @functools.partial(
      pl.kernel,
      out_type=jax.ShapeDtypeStruct((num_rows, EMB_DIM), jnp.float32),
      mesh=mesh,
      scratch_types=scratch,
      compiler_params=pltpu.CompilerParams(use_tc_tiling_on_sc=False),
  )
  def gather_kernel(table_hbm, idx_hbm, out_hbm, idx_v, *rest):
    bufs = rest[:NBUF]
    gsems = rest[NBUF:2 * NBUF]
    wsems = rest[2 * NBUF:]
    wid = lax.axis_index("s") * NC + lax.axis_index("c")
    base = wid * per_w
    pltpu.sync_copy(idx_hbm.at[wid], idx_v)

    def issue_gathers(s, b):
      for j in range(K):
        pltpu.async_copy(
            table_hbm.at[idx_v.at[s * K + j]],
            bufs[b].at[pl.ds(j * CHUNK, CHUNK)],
            gsems[b])

    for b in range(NBUF):
      issue_gathers(jnp.int32(b), b)

    def group(gi, carry):
      s0 = gi * NBUF
      for b in range(NBUF):
        s = s0 + b
        pltpu.make_async_copy(table_hbm.at[pl.ds(0, SUPER)], bufs[b],
                              gsems[b]).wait()
        wr = pltpu.make_async_copy(
            bufs[b], out_hbm.at[pl.ds(base + s * SUPER, SUPER)], wsems[b])
        wr.start()
        wr.wait()

        @pl.when(gi < n_groups - 1)
        def _():
          issue_gathers(s + NBUF, b)

      return carry

    lax.fori_loop(0, n_groups, group, 0)

  return gather_kernel


def kernel(token_ids, embedding_matrix):
  batch, hist = token_ids.shape
  total = batch * hist
  per_w = total // NW
  n_chunks = per_w // CHUNK
  idx = token_ids.astype(jnp.int32).reshape(NW, n_chunks, CHUNK)
  out = _make_gather(total, n_chunks)(embedding_matrix, idx)
  return out.reshape(batch, hist, EMB_DIM)
